# baseline (speedup 1.0000x reference)
import functools

import jax
import jax.numpy as jnp
from jax import lax
from jax.experimental import pallas as pl
from jax.experimental.pallas import tpu as pltpu
from jax.experimental.pallas import tpu_sc as plsc

N_NODES = 10000
N_FEAT = 128
HID = 16
N_EDGES = 320000
SLOPE = 0.01

NC = 2
NS = 16
NW = NC * NS
EPT = N_EDGES // NW
BROWS = 320
HALF = NS * BROWS
TRASH = HALF
CH = 80
REG = 10320
L = 16


def _leaky(v):
    return jnp.where(v >= 0, v, SLOPE * v)


def _bin_body(src_hbm, dst_hbm, gsrc_hbm, gdl_hbm, cnt_hbm,
              src_v, dst_v, hist_v, cur_v, ssrc_v, sdl_v):
    c = lax.axis_index("c")
    s = lax.axis_index("s")
    wid = c * NS + s

    pltpu.sync_copy(src_hbm.at[wid], src_v)
    pltpu.sync_copy(dst_hbm.at[wid], dst_v)

    hist_v[pl.ds(0, L)] = jnp.zeros((L,), jnp.int32)
    hist_v[pl.ds(L, L)] = jnp.zeros((L,), jnp.int32)

    def _count(v, carry):
        d = dst_v[pl.ds(v * L, L)]
        k = d // BROWS
        rank, lastm = plsc.scan_count(k)
        h = plsc.load_gather(hist_v, [k])
        plsc.store_scatter(hist_v, [k], h + rank, mask=lastm)
        return carry
    lax.fori_loop(0, EPT // L, _count, 0)

    h0 = hist_v[pl.ds(0, L)]
    h1 = hist_v[pl.ds(L, L)]
    c80 = ((h0 + 7) // 8) * 8
    c81 = ((h1 + 7) // 8) * 8
    cur_v[pl.ds(0, L)] = plsc.cumsum(c80) - c80
    cur_v[pl.ds(L, L)] = plsc.cumsum(c81) - c81 + jnp.sum(c80)

    def _dist(v, carry):
        d = dst_v[pl.ds(v * L, L)]
        sv = src_v[pl.ds(v * L, L)]
        k = d // BROWS
        dl = d - HALF * (k // NS)
        rank, lastm = plsc.scan_count(k)
        cbase = plsc.load_gather(cur_v, [k])
        pos = cbase + rank - 1
        plsc.store_scatter(ssrc_v, [pos], sv)
        plsc.store_scatter(sdl_v, [pos], dl)
        plsc.store_scatter(cur_v, [k], cbase + rank,
                           mask=lastm)
        return carry
    lax.fori_loop(0, EPT // L, _dist, 0)

    pltpu.sync_copy(ssrc_v, gsrc_hbm.at[wid])
    pltpu.sync_copy(sdl_v, gdl_hbm.at[wid])
    pltpu.sync_copy(hist_v, cnt_hbm.at[wid])


@functools.partial(
    pl.kernel,
    out_type=[
        jax.ShapeDtypeStruct((NW, REG), jnp.int32),
        jax.ShapeDtypeStruct((NW, REG), jnp.int32),
        jax.ShapeDtypeStruct((NW, NW), jnp.int32),
    ],
    mesh=plsc.VectorSubcoreMesh(core_axis_name="c", subcore_axis_name="s"),
    scratch_types=[
        pltpu.VMEM((EPT,), jnp.int32),
        pltpu.VMEM((EPT,), jnp.int32),
        pltpu.VMEM((NW,), jnp.int32),
        pltpu.VMEM((NW,), jnp.int32),
        pltpu.VMEM((REG,), jnp.int32),
        pltpu.VMEM((REG,), jnp.int32),
    ],
    compiler_params=pltpu.CompilerParams(use_tc_tiling_on_sc=False,
                                         needs_layout_passes=False),
)
def _bin_edges(src_hbm, dst_hbm, gsrc_hbm, gdl_hbm, cnt_hbm, *scr):
    _bin_body(src_hbm, dst_hbm, gsrc_hbm, gdl_hbm, cnt_hbm, *scr)


def _seg_body(D, table_hbm, gsrc_hbm, gdl_hbm, cnt_hbm, zz_hbm, out_hbm,
              cnt_v, si, di, rows, acc_sh, sem):
    c = lax.axis_index("c")
    s = lax.axis_index("s")
    k = c * NS + s
    my_lo = s * BROWS

    pltpu.sync_copy(cnt_hbm, cnt_v)
    pltpu.sync_copy(zz_hbm, acc_sh.at[pl.ds(my_lo, BROWS)])

    lanes = lax.iota(jnp.int32, L)
    imin = jnp.int32(-2147483648)

    def _cscal(idx):
        grp = pl.multiple_of((idx // L) * L, L)
        v = cnt_v[pl.ds(grp, L)]
        return jnp.max(jnp.where(lanes == idx - grp, v, imin))

    def _run(t, carry):
        cnt = _cscal(t * NW + k)

        def _loff(kk, o):
            return o + ((_cscal(t * NW + kk) + 7) // 8) * 8
        base = t * REG + lax.fori_loop(0, k, _loff, 0)

        def _chunk(j, cc):
            off = pl.multiple_of(base + CH * j, 8)
            valid = jnp.minimum(cnt - CH * j, CH)
            pltpu.sync_copy(gsrc_hbm.at[pl.ds(off, CH)], si)
            pltpu.sync_copy(gdl_hbm.at[pl.ds(off, CH)], di)
            for g in range(CH // L):
                bad = (lanes + g * L) >= valid
                sg = si[pl.ds(g * L, L)]
                dg = di[pl.ds(g * L, L)]
                si[pl.ds(g * L, L)] = jnp.where(bad, 0, sg)
                di[pl.ds(g * L, L)] = jnp.where(bad, TRASH, dg)
            pltpu.async_copy(table_hbm.at[si], rows, sem).wait()
            pltpu.sync_copy(rows, acc_sh.at[di], add=True)
            return cc
        return lax.fori_loop(0, (cnt + CH - 1) // CH, _chunk, carry)
    lax.fori_loop(0, NW, _run, 0)

    pltpu.sync_copy(acc_sh.at[pl.ds(my_lo, BROWS)],
                    out_hbm.at[pl.ds(k * BROWS, BROWS)])


def _make_seg(D):
    @functools.partial(
        pl.kernel,
        out_type=jax.ShapeDtypeStruct((NW * BROWS, D), jnp.float32),
        mesh=plsc.VectorSubcoreMesh(core_axis_name="c",
                                    subcore_axis_name="s"),
        scratch_types=[
            pltpu.VMEM((NW * NW,), jnp.int32),
            pltpu.VMEM((CH,), jnp.int32),
            pltpu.VMEM((CH,), jnp.int32),
            pltpu.VMEM((CH, D), jnp.float32),
            pltpu.VMEM_SHARED((HALF + 8, D), jnp.float32),
            pltpu.SemaphoreType.DMA,
        ],
        compiler_params=pltpu.CompilerParams(use_tc_tiling_on_sc=False,
                                             needs_layout_passes=False),
    )
    def _seg(table_hbm, gsrc_hbm, gdl_hbm, cnt_hbm, zz_hbm, out_hbm, *scr):
        _seg_body(D, table_hbm, gsrc_hbm, gdl_hbm, cnt_hbm, zz_hbm, out_hbm,
                  *scr)

    return _seg


_seg128 = _make_seg(N_FEAT)
_seg16 = _make_seg(HID)


def _layer_body(h_ref, a_ref, w1_ref, b1_ref, w2_ref, b2_ref, o_ref, o2_ref):
    hh = h_ref[...] + a_ref[...]
    hh = _leaky(jnp.dot(hh, w1_ref[...],
                        preferred_element_type=jnp.float32) + b1_ref[...])
    hh = jnp.dot(hh, w2_ref[...],
                 preferred_element_type=jnp.float32) + b2_ref[...]
    out = _leaky(hh)
    o_ref[...] = out
    o2_ref[...] = out


def _layer(h, aggr, w1, b1, w2, b2):
    return pl.pallas_call(
        _layer_body,
        out_shape=[jax.ShapeDtypeStruct((N_NODES, HID), jnp.float32)] * 2,
    )(h, aggr, w1, b1.reshape(1, HID), w2, b2.reshape(1, HID))


def _head_body(h_ref, a_ref, w1_ref, b1_ref, w2_ref, b2_ref, wfc_ref,
               bfc_ref, sc_ref, o_ref):
    hh = h_ref[...] + a_ref[...]
    hh = _leaky(jnp.dot(hh, w1_ref[...],
                        preferred_element_type=jnp.float32) + b1_ref[...])
    h3 = jnp.dot(hh, w2_ref[...],
                 preferred_element_type=jnp.float32) + b2_ref[...]
    h3 = h3 * sc_ref[...]
    o = jnp.dot(_leaky(h3), wfc_ref[...],
                preferred_element_type=jnp.float32) + bfc_ref[...]
    o_ref[...] = o[0:1, :]


def _head(h, aggr, w1, b1, w2, b2, wfc, bfc, scale):
    return pl.pallas_call(
        _head_body,
        out_shape=jax.ShapeDtypeStruct((1, 1), jnp.float32),
    )(h, aggr, w1, b1.reshape(1, HID), w2, b2.reshape(1, HID), wfc,
      bfc.reshape(1, 1), scale)


def kernel(x, edge_index, batch_size,
           W1_1, b1_1, W2_1, b2_1,
           W1_2, b1_2, W2_2, b2_2,
           W1_3, b1_3, W2_3, b2_3,
           Wfc, bfc):
    src = edge_index[0].reshape(NW, EPT)
    dst = edge_index[1].reshape(NW, EPT)

    gsrc, gdl, counts = _bin_edges(src, dst)
    gsrc = gsrc.reshape(NW * REG)
    gdl = gdl.reshape(NW * REG)
    counts = counts.reshape(NW * NW)
    z128 = jnp.zeros((BROWS, N_FEAT), jnp.float32)
    z16 = jnp.zeros((BROWS, HID), jnp.float32)

    a1 = _seg128(x, gsrc, gdl, counts, z128)[:N_NODES]
    h1, h1_sc = _layer(x, a1, W1_1, b1_1, W2_1, b2_1)
    a2 = _seg16(h1_sc, gsrc, gdl, counts, z16)[:N_NODES]
    h2, h2_sc = _layer(h1, a2, W1_2, b1_2, W2_2, b2_2)
    a3 = _seg16(h2_sc, gsrc, gdl, counts, z16)[:N_NODES]

    scale = (jnp.asarray(batch_size) // 1).astype(jnp.float32).reshape(1, 1)
    o = _head(h2, a3, W1_3, b1_3, W2_3, b2_3, Wfc, bfc, scale)
    return o.reshape(())
